# baseline (speedup 1.0000x reference)
import functools

import jax
import jax.numpy as jnp
import numpy as np
from jax import lax
from jax.experimental import pallas as pl
from jax.experimental.pallas import tpu as pltpu
from jax.experimental.pallas import tpu_sc as plsc

N = 10000
E = 160000
D_IN = 128
H1 = 64
H2 = 32
EMB = 16

NC = 2
NS = 16
NW = NC * NS

N_PAD = N + 112
ROWS_PER_TILE = N_PAD // NS
DEG_M = 2 * N + 16
MP_CHUNK = 128
MP_NCHUNK0 = 56
MP_NCHUNK1 = 24
NCHUNKS = NS * (MP_NCHUNK0 + MP_NCHUNK1)
DEG_VEC0 = 832
DEG_VEC1 = 448


def _stage_a_body(f_ref, w1_ref, o_ref):
    o_ref[...] = jnp.dot(f_ref[...], w1_ref[...],
                         preferred_element_type=jnp.float32)


def _stage_a(features, W1):
    blk = 1000
    return pl.pallas_call(
        _stage_a_body,
        grid=(N // blk,),
        in_specs=[
            pl.BlockSpec((blk, D_IN), lambda i: (i, 0)),
            pl.BlockSpec((D_IN, H1), lambda i: (0, 0)),
        ],
        out_specs=pl.BlockSpec((blk, H1), lambda i: (i, 0)),
        out_shape=jax.ShapeDtypeStruct((N, H1), jnp.float32),
    )(features, W1)


def _scale_body(z_ref, ns_ref, o_ref):
    o_ref[...] = (z_ref[...] * ns_ref[...]).astype(jnp.bfloat16)


def _scale(z1, norm_src):
    blk = 1000
    return pl.pallas_call(
        _scale_body,
        grid=(N // blk,),
        in_specs=[
            pl.BlockSpec((blk, H1), lambda i: (i, 0)),
            pl.BlockSpec((blk, 1), lambda i: (i, 0)),
        ],
        out_specs=pl.BlockSpec((blk, H1), lambda i: (i, 0)),
        out_shape=jax.ShapeDtypeStruct((N, H1), jnp.bfloat16),
    )(z1, norm_src)


def _stage_b_body(m_ref, nd_ref, b1_ref, w2_ref, wext_ref, ns_ref, o_ref):
    m = m_ref[0].astype(jnp.float32) + m_ref[1].astype(jnp.float32)
    x = jnp.maximum(m * nd_ref[...] + b1_ref[...], 0.0)
    w2e = jnp.dot(w2_ref[...], wext_ref[...], preferred_element_type=jnp.float32)
    o_ref[...] = jnp.dot(x, w2e, preferred_element_type=jnp.float32) * ns_ref[...]


def _stage_b(msg1p, norm_dst, b1, W2, Wext, norm_src):
    blk = 1000
    return pl.pallas_call(
        _stage_b_body,
        grid=(N // blk,),
        in_specs=[
            pl.BlockSpec((NC, blk, H1), lambda i: (0, i, 0)),
            pl.BlockSpec((blk, 1), lambda i: (i, 0)),
            pl.BlockSpec((1, H1), lambda i: (0, 0)),
            pl.BlockSpec((H1, H2), lambda i: (0, 0)),
            pl.BlockSpec((H2, EMB), lambda i: (0, 0)),
            pl.BlockSpec((blk, 1), lambda i: (i, 0)),
        ],
        out_specs=pl.BlockSpec((blk, EMB), lambda i: (i, 0)),
        out_shape=jax.ShapeDtypeStruct((N, EMB), jnp.float32),
    )(msg1p, norm_dst, b1, W2, Wext, norm_src)


def _stage_c1_body(m_ref, nd_ref, b2_ref, wext_ref, bext_ref, o_ref):
    b2e = jnp.dot(b2_ref[...], wext_ref[...], preferred_element_type=jnp.float32)
    o_ref[...] = (m_ref[0] + m_ref[1]) * nd_ref[...] + b2e + bext_ref[...]


def _stage_c1(msg2p, norm_dst, b2, Wext, bext):
    blk = 2000
    return pl.pallas_call(
        _stage_c1_body,
        grid=(N // blk,),
        in_specs=[
            pl.BlockSpec((NC, blk, EMB), lambda i: (0, i, 0)),
            pl.BlockSpec((blk, 1), lambda i: (i, 0)),
            pl.BlockSpec((1, H2), lambda i: (0, 0)),
            pl.BlockSpec((H2, EMB), lambda i: (0, 0)),
            pl.BlockSpec((1, EMB), lambda i: (0, 0)),
        ],
        out_specs=pl.BlockSpec((blk, EMB), lambda i: (i, 0)),
        out_shape=jax.ShapeDtypeStruct((N, EMB), jnp.float32),
    )(msg2p, norm_dst, b2, Wext, bext)


def _stage_c2_body(ei_ref, ej_ref, o_ref):
    g = lax.dot_general(ei_ref[...], ej_ref[...],
                        (((1,), (1,)), ((), ())),
                        preferred_element_type=jnp.float32)
    o_ref[...] = jax.nn.sigmoid(g)


def _stage_c2(emb):
    bi, bj = 512, 1024
    gi = (N + bi - 1) // bi
    gj = (N + bj - 1) // bj
    return pl.pallas_call(
        _stage_c2_body,
        grid=(gi, gj),
        in_specs=[
            pl.BlockSpec((bi, EMB), lambda i, j: (i, 0)),
            pl.BlockSpec((bj, EMB), lambda i, j: (j, 0)),
        ],
        out_specs=pl.BlockSpec((bi, bj), lambda i, j: (i, j)),
        out_shape=jax.ShapeDtypeStruct((N, N), jnp.float32),
    )(emb, emb)


def _norms_body(dp_ref, o_ref):
    deg = jnp.sum(dp_ref[...], axis=0, keepdims=True)
    o_ref[...] = lax.rsqrt(jnp.maximum(deg, 1.0))


def _norms(deg_partials):
    p, m = deg_partials.shape
    return pl.pallas_call(
        _norms_body,
        in_specs=[pl.BlockSpec((p, m), lambda: (0, 0))],
        out_specs=pl.BlockSpec((1, m), lambda: (0, 0)),
        out_shape=jax.ShapeDtypeStruct((1, m), jnp.float32),
    )(deg_partials)


_SC_MESH = plsc.VectorSubcoreMesh(core_axis_name="c", subcore_axis_name="s")
_SC_PARAMS = pltpu.CompilerParams(needs_layout_passes=False,
                                  use_tc_tiling_on_sc=False)


@functools.partial(
    pl.kernel,
    out_type=jax.ShapeDtypeStruct((NW, DEG_M), jnp.float32),
    mesh=_SC_MESH,
    compiler_params=_SC_PARAMS,
    scratch_types=[
        pltpu.VMEM((DEG_VEC0 * 16,), jnp.int32),
        pltpu.VMEM((DEG_VEC1 * 16,), jnp.int32),
        pltpu.VMEM((DEG_M,), jnp.float32),
    ],
)
def _sc_degrees(idx0_hbm, idx1_hbm, out_hbm, idx0_v, idx1_v, acc_v):
    c = lax.axis_index("c")
    s = lax.axis_index("s")
    wid = s * NC + c
    pltpu.sync_copy(idx0_hbm.at[s], idx0_v)
    pltpu.sync_copy(idx1_hbm.at[s], idx1_v)
    zeros16 = jnp.zeros((16,), jnp.float32)

    def zbody(i, carry):
        acc_v[pl.ds(i * 16, 16)] = zeros16
        return carry

    lax.fori_loop(0, DEG_M // 16, zbody, 0)
    ones16 = jnp.ones((16,), jnp.float32)

    def ebody0(i, carry):
        v = idx0_v[pl.ds(i * 16, 16)]
        plsc.addupdate_scatter(acc_v, [v], ones16)
        return carry

    def ebody1(i, carry):
        v = idx1_v[pl.ds(i * 16, 16)]
        plsc.addupdate_scatter(acc_v, [v], ones16)
        return carry

    lax.fori_loop(0, jnp.where(c == 0, 0, DEG_VEC0), ebody0, 0)
    lax.fori_loop(0, jnp.where(c == 0, DEG_VEC1, 0), ebody1, 0)
    pltpu.sync_copy(acc_v, out_hbm.at[wid])


def _make_sc_mp(W, dtype):
    nbuf = 8
    lanes = 16 if dtype == jnp.float32 else 32

    @functools.partial(
        pl.kernel,
        out_type=jax.ShapeDtypeStruct((NC, N_PAD, W), dtype),
        mesh=_SC_MESH,
        compiler_params=_SC_PARAMS,
        scratch_types=[
            pltpu.VMEM((MP_NCHUNK0, MP_CHUNK), jnp.int32),
            pltpu.VMEM((MP_NCHUNK0, MP_CHUNK), jnp.int32),
            pltpu.VMEM((MP_NCHUNK1, MP_CHUNK), jnp.int32),
            pltpu.VMEM((MP_NCHUNK1, MP_CHUNK), jnp.int32),
            pltpu.VMEM((nbuf, MP_CHUNK, W), dtype),
            pltpu.VMEM_SHARED((N_PAD, W), dtype),
        ] + [pltpu.SemaphoreType.DMA] * (2 * nbuf),
    )
    def mp(z_hbm, src0_hbm, src1_hbm, dst0_hbm, dst1_hbm, out_hbm,
           src0_v, dst0_v, src1_v, dst1_v, gbuf, acc_sh, *sems):
        gsem = sems[:nbuf]
        ssem = sems[nbuf:]
        c = lax.axis_index("c")
        s = lax.axis_index("s")
        pltpu.sync_copy(src0_hbm.at[s], src0_v)
        pltpu.sync_copy(dst0_hbm.at[s], dst0_v)
        pltpu.sync_copy(src1_hbm.at[s], src1_v)
        pltpu.sync_copy(dst1_hbm.at[s], dst1_v)
        zvec = jnp.zeros((lanes,), dtype)
        wv = W // lanes

        def zbody(i, carry):
            gbuf[0, i // wv, pl.ds((i % wv) * lanes, lanes)] = zvec
            return carry

        lax.fori_loop(0, MP_CHUNK * wv, zbody, 0)
        row0 = s * ROWS_PER_TILE
        chunks = []
        off = 0
        while off < ROWS_PER_TILE:
            sz = min(MP_CHUNK, ROWS_PER_TILE - off)
            chunks.append((off, sz))
            off += sz
        for off, sz in chunks:
            pltpu.sync_copy(gbuf.at[0].at[pl.ds(0, sz)],
                            acc_sh.at[pl.ds(row0 + off, sz)])
        plsc.subcore_barrier()

        def gather(cid, b, src_v):
            return pltpu.async_copy(z_hbm.at[src_v.at[cid]], gbuf.at[b],
                                    gsem[b])

        def scatter(cid, b, dst_v):
            return pltpu.async_copy(gbuf.at[b], acc_sh.at[dst_v.at[cid]],
                                    ssem[b], add=True)

        look = 4

        def pipeline(nchunk, src_v, dst_v):
            gdesc = {cc: gather(cc, cc % nbuf, src_v) for cc in range(look)}
            sdesc = {}
            for cid in range(nchunk):
                gdesc[cid].wait()
                sdesc[cid] = scatter(cid, cid % nbuf, dst_v)
                nxt = cid + look
                if nxt < nchunk:
                    if nxt - nbuf in sdesc:
                        sdesc[nxt - nbuf].wait()
                    gdesc[nxt] = gather(nxt, nxt % nbuf, src_v)
            for cid in range(nchunk - nbuf, nchunk):
                sdesc[cid].wait()

        @pl.when(c != 0)
        def _():
            pipeline(MP_NCHUNK0, src0_v, dst0_v)

        @pl.when(c == 0)
        def _():
            pipeline(MP_NCHUNK1, src1_v, dst1_v)

        plsc.subcore_barrier()
        for off, sz in chunks:
            pltpu.sync_copy(acc_sh.at[pl.ds(row0 + off, sz)],
                            gbuf.at[0].at[pl.ds(0, sz)])
            pltpu.sync_copy(gbuf.at[0].at[pl.ds(0, sz)],
                            out_hbm.at[c].at[pl.ds(row0 + off, sz)])

    return mp


_sc_mp64 = _make_sc_mp(H1, jnp.bfloat16)
_sc_mp16 = _make_sc_mp(EMB, jnp.float32)


def kernel(features, edge_index, W1, b1, W2, b2, Wext, bext):
    src, dst = edge_index[0], edge_index[1]

    n0 = NS * MP_NCHUNK0 * MP_CHUNK
    pad_e = NCHUNKS * MP_CHUNK - E
    srcf = jnp.concatenate([src, jnp.zeros((pad_e,), jnp.int32)])
    dstf = jnp.concatenate([dst, jnp.full((pad_e,), N, jnp.int32)])
    src_p0 = srcf[:n0].reshape(NS, MP_NCHUNK0, MP_CHUNK)
    src_p1 = srcf[n0:].reshape(NS, MP_NCHUNK1, MP_CHUNK)
    dst_p0 = dstf[:n0].reshape(NS, MP_NCHUNK0, MP_CHUNK)
    dst_p1 = dstf[n0:].reshape(NS, MP_NCHUNK1, MP_CHUNK)
    nvec_tot = NS * (DEG_VEC0 + DEG_VEC1)
    d0 = NS * DEG_VEC0 * 16
    degf = jnp.concatenate(
        [src, dst + N, jnp.full((nvec_tot * 16 - 2 * E,), 2 * N, jnp.int32)])
    deg_idx0 = degf[:d0].reshape(NS, DEG_VEC0 * 16)
    deg_idx1 = degf[d0:].reshape(NS, DEG_VEC1 * 16)

    deg_partials = _sc_degrees(deg_idx0, deg_idx1)
    norms = _norms(deg_partials)[0]
    norm_src = norms[:N].reshape(N, 1)
    norm_dst = norms[N:2 * N].reshape(N, 1)

    z1 = _stage_a(features, W1)
    z1s = _scale(z1, norm_src)
    p1 = _sc_mp64(z1s, src_p0, src_p1, dst_p0, dst_p1)
    z2s = _stage_b(p1, norm_dst, b1.reshape(1, H1), W2, Wext, norm_src)
    p2 = _sc_mp16(z2s, src_p0, src_p1, dst_p0, dst_p1)
    emb_long = _stage_c1(p2, norm_dst, b2.reshape(1, H2), Wext,
                         bext.reshape(1, EMB))
    logits = _stage_c2(emb_long)
    return (emb_long, logits)


# speedup vs baseline: 1.0733x; 1.0733x over previous
import functools

import jax
import jax.numpy as jnp
import numpy as np
from jax import lax
from jax.experimental import pallas as pl
from jax.experimental.pallas import tpu as pltpu
from jax.experimental.pallas import tpu_sc as plsc

N = 10000
E = 160000
D_IN = 128
H1 = 64
H2 = 32
EMB = 16

NC = 2
NS = 16
NW = NC * NS

N_PAD = N + 112
ROWS_PER_TILE = N_PAD // NS
DEG_M = 2 * N + 16
MP_CHUNK = 128
MP_NCHUNK = 40
NCHUNKS = NW * MP_NCHUNK
DEG_VEC = 640


def _stage_a_body(f_ref, w1_ref, o_ref):
    o_ref[...] = jnp.dot(f_ref[...], w1_ref[...],
                         preferred_element_type=jnp.float32)


def _stage_a(features, W1):
    blk = 1000
    return pl.pallas_call(
        _stage_a_body,
        grid=(N // blk,),
        in_specs=[
            pl.BlockSpec((blk, D_IN), lambda i: (i, 0)),
            pl.BlockSpec((D_IN, H1), lambda i: (0, 0)),
        ],
        out_specs=pl.BlockSpec((blk, H1), lambda i: (i, 0)),
        out_shape=jax.ShapeDtypeStruct((N, H1), jnp.float32),
    )(features, W1)


def _scale_body(z_ref, ns_ref, o_ref):
    o_ref[...] = (z_ref[...] * ns_ref[...]).astype(jnp.bfloat16)


def _scale(z1, norm_src):
    blk = 1000
    return pl.pallas_call(
        _scale_body,
        grid=(N // blk,),
        in_specs=[
            pl.BlockSpec((blk, H1), lambda i: (i, 0)),
            pl.BlockSpec((blk, 1), lambda i: (i, 0)),
        ],
        out_specs=pl.BlockSpec((blk, H1), lambda i: (i, 0)),
        out_shape=jax.ShapeDtypeStruct((N, H1), jnp.bfloat16),
    )(z1, norm_src)


def _stage_b_body(m_ref, nd_ref, b1_ref, w2_ref, wext_ref, ns_ref, o_ref):
    m = m_ref[0].astype(jnp.float32) + m_ref[1].astype(jnp.float32)
    x = jnp.maximum(m * nd_ref[...] + b1_ref[...], 0.0)
    w2e = jnp.dot(w2_ref[...], wext_ref[...], preferred_element_type=jnp.float32)
    o_ref[...] = jnp.dot(x, w2e, preferred_element_type=jnp.float32) * ns_ref[...]


def _stage_b(msg1p, norm_dst, b1, W2, Wext, norm_src):
    blk = 1000
    return pl.pallas_call(
        _stage_b_body,
        grid=(N // blk,),
        in_specs=[
            pl.BlockSpec((NC, blk, H1), lambda i: (0, i, 0)),
            pl.BlockSpec((blk, 1), lambda i: (i, 0)),
            pl.BlockSpec((1, H1), lambda i: (0, 0)),
            pl.BlockSpec((H1, H2), lambda i: (0, 0)),
            pl.BlockSpec((H2, EMB), lambda i: (0, 0)),
            pl.BlockSpec((blk, 1), lambda i: (i, 0)),
        ],
        out_specs=pl.BlockSpec((blk, EMB), lambda i: (i, 0)),
        out_shape=jax.ShapeDtypeStruct((N, EMB), jnp.float32),
    )(msg1p, norm_dst, b1, W2, Wext, norm_src)


def _stage_c1_body(m_ref, nd_ref, b2_ref, wext_ref, bext_ref, o_ref):
    b2e = jnp.dot(b2_ref[...], wext_ref[...], preferred_element_type=jnp.float32)
    o_ref[...] = (m_ref[0] + m_ref[1]) * nd_ref[...] + b2e + bext_ref[...]


def _stage_c1(msg2p, norm_dst, b2, Wext, bext):
    blk = 2000
    return pl.pallas_call(
        _stage_c1_body,
        grid=(N // blk,),
        in_specs=[
            pl.BlockSpec((NC, blk, EMB), lambda i: (0, i, 0)),
            pl.BlockSpec((blk, 1), lambda i: (i, 0)),
            pl.BlockSpec((1, H2), lambda i: (0, 0)),
            pl.BlockSpec((H2, EMB), lambda i: (0, 0)),
            pl.BlockSpec((1, EMB), lambda i: (0, 0)),
        ],
        out_specs=pl.BlockSpec((blk, EMB), lambda i: (i, 0)),
        out_shape=jax.ShapeDtypeStruct((N, EMB), jnp.float32),
    )(msg2p, norm_dst, b2, Wext, bext)


def _stage_c2_body(ei_ref, ej_ref, o_ref):
    g = lax.dot_general(ei_ref[...], ej_ref[...],
                        (((1,), (1,)), ((), ())),
                        preferred_element_type=jnp.float32)
    o_ref[...] = 0.5 * jnp.tanh(0.5 * g) + 0.5


def _stage_c2(emb):
    bi, bj = 512, 1024
    gi = (N + bi - 1) // bi
    gj = (N + bj - 1) // bj
    return pl.pallas_call(
        _stage_c2_body,
        grid=(gi, gj),
        in_specs=[
            pl.BlockSpec((bi, EMB), lambda i, j: (i, 0)),
            pl.BlockSpec((bj, EMB), lambda i, j: (j, 0)),
        ],
        out_specs=pl.BlockSpec((bi, bj), lambda i, j: (i, j)),
        out_shape=jax.ShapeDtypeStruct((N, N), jnp.float32),
    )(emb, emb)


def _norms_body(dp_ref, o_ref):
    deg = jnp.sum(dp_ref[...], axis=0, keepdims=True)
    o_ref[...] = lax.rsqrt(jnp.maximum(deg, 1.0))


def _norms(deg_partials):
    p, m = deg_partials.shape
    return pl.pallas_call(
        _norms_body,
        in_specs=[pl.BlockSpec((p, m), lambda: (0, 0))],
        out_specs=pl.BlockSpec((1, m), lambda: (0, 0)),
        out_shape=jax.ShapeDtypeStruct((1, m), jnp.float32),
    )(deg_partials)


_SC_MESH = plsc.VectorSubcoreMesh(core_axis_name="c", subcore_axis_name="s")
_SC_PARAMS = pltpu.CompilerParams(needs_layout_passes=False,
                                  use_tc_tiling_on_sc=False)


@functools.partial(
    pl.kernel,
    out_type=jax.ShapeDtypeStruct((NW, DEG_M), jnp.float32),
    mesh=_SC_MESH,
    compiler_params=_SC_PARAMS,
    scratch_types=[
        pltpu.VMEM((DEG_VEC * 16,), jnp.int32),
        pltpu.VMEM((DEG_M,), jnp.float32),
    ],
)
def _sc_degrees(idx_hbm, out_hbm, idx_v, acc_v):
    c = lax.axis_index("c")
    s = lax.axis_index("s")
    wid = s * NC + c
    pltpu.sync_copy(idx_hbm.at[c].at[s], idx_v)
    zeros16 = jnp.zeros((16,), jnp.float32)

    def zbody(i, carry):
        acc_v[pl.ds(i * 16, 16)] = zeros16
        return carry

    lax.fori_loop(0, DEG_M // 16, zbody, 0)
    ones16 = jnp.ones((16,), jnp.float32)

    def ebody(i, carry):
        v = idx_v[pl.ds(i * 16, 16)]
        plsc.addupdate_scatter(acc_v, [v], ones16)
        return carry

    lax.fori_loop(0, DEG_VEC, ebody, 0)
    pltpu.sync_copy(acc_v, out_hbm.at[wid])


def _make_sc_mp(W, dtype):
    nbuf = 8
    lanes = 16 if dtype == jnp.float32 else 32

    @functools.partial(
        pl.kernel,
        out_type=jax.ShapeDtypeStruct((NC, N_PAD, W), dtype),
        mesh=_SC_MESH,
        compiler_params=_SC_PARAMS,
        scratch_types=[
            pltpu.VMEM((MP_NCHUNK, MP_CHUNK), jnp.int32),
            pltpu.VMEM((MP_NCHUNK, MP_CHUNK), jnp.int32),
            pltpu.VMEM((nbuf, MP_CHUNK, W), dtype),
            pltpu.VMEM_SHARED((N_PAD, W), dtype),
        ] + [pltpu.SemaphoreType.DMA] * (2 * nbuf),
    )
    def mp(z_hbm, src_hbm, dst_hbm, out_hbm,
           src_v, dst_v, gbuf, acc_sh, *sems):
        gsem = sems[:nbuf]
        ssem = sems[nbuf:]
        c = lax.axis_index("c")
        s = lax.axis_index("s")
        pltpu.sync_copy(src_hbm.at[c].at[s], src_v)
        pltpu.sync_copy(dst_hbm.at[c].at[s], dst_v)
        zvec = jnp.zeros((lanes,), dtype)
        wv = W // lanes

        def zbody(i, carry):
            gbuf[0, i // wv, pl.ds((i % wv) * lanes, lanes)] = zvec
            return carry

        lax.fori_loop(0, MP_CHUNK * wv, zbody, 0)
        row0 = s * ROWS_PER_TILE
        chunks = []
        off = 0
        while off < ROWS_PER_TILE:
            sz = min(MP_CHUNK, ROWS_PER_TILE - off)
            chunks.append((off, sz))
            off += sz
        for off, sz in chunks:
            pltpu.sync_copy(gbuf.at[0].at[pl.ds(0, sz)],
                            acc_sh.at[pl.ds(row0 + off, sz)])
        plsc.subcore_barrier()

        def gather(cid, b, src_v):
            return pltpu.async_copy(z_hbm.at[src_v.at[cid]], gbuf.at[b],
                                    gsem[b])

        def scatter(cid, b, dst_v):
            return pltpu.async_copy(gbuf.at[b], acc_sh.at[dst_v.at[cid]],
                                    ssem[b], add=True)

        look = 4

        nchunk = MP_NCHUNK
        gdesc = {cc: gather(cc, cc % nbuf, src_v) for cc in range(look)}
        sdesc = {}
        for cid in range(nchunk):
            gdesc[cid].wait()
            sdesc[cid] = scatter(cid, cid % nbuf, dst_v)
            nxt = cid + look
            if nxt < nchunk:
                if nxt - nbuf in sdesc:
                    sdesc[nxt - nbuf].wait()
                gdesc[nxt] = gather(nxt, nxt % nbuf, src_v)
        for cid in range(nchunk - nbuf, nchunk):
            sdesc[cid].wait()
        plsc.subcore_barrier()
        for off, sz in chunks:
            pltpu.sync_copy(acc_sh.at[pl.ds(row0 + off, sz)],
                            gbuf.at[0].at[pl.ds(0, sz)])
            pltpu.sync_copy(gbuf.at[0].at[pl.ds(0, sz)],
                            out_hbm.at[c].at[pl.ds(row0 + off, sz)])

    return mp


_sc_mp64 = _make_sc_mp(H1, jnp.bfloat16)
_sc_mp16 = _make_sc_mp(EMB, jnp.float32)


def kernel(features, edge_index, W1, b1, W2, b2, Wext, bext):
    src, dst = edge_index[0], edge_index[1]

    pad_e = NCHUNKS * MP_CHUNK - E
    srcf = jnp.concatenate([src, jnp.zeros((pad_e,), jnp.int32)])
    dstf = jnp.concatenate([dst, jnp.full((pad_e,), N, jnp.int32)])
    src_p = srcf.reshape(NC, NS, MP_NCHUNK, MP_CHUNK)
    dst_p = dstf.reshape(NC, NS, MP_NCHUNK, MP_CHUNK)
    nvec_tot = NW * DEG_VEC
    degf = jnp.concatenate(
        [src, dst + N, jnp.full((nvec_tot * 16 - 2 * E,), 2 * N, jnp.int32)])
    deg_idx = degf.reshape(NC, NS, DEG_VEC * 16)

    deg_partials = _sc_degrees(deg_idx)
    norms = _norms(deg_partials)[0]
    norm_src = norms[:N].reshape(N, 1)
    norm_dst = norms[N:2 * N].reshape(N, 1)

    z1 = _stage_a(features, W1)
    z1s = _scale(z1, norm_src)
    p1 = _sc_mp64(z1s, src_p, dst_p)
    z2s = _stage_b(p1, norm_dst, b1.reshape(1, H1), W2, Wext, norm_src)
    p2 = _sc_mp16(z2s, src_p, dst_p)
    emb_long = _stage_c1(p2, norm_dst, b2.reshape(1, H2), Wext,
                         bext.reshape(1, EMB))
    logits = _stage_c2(emb_long)
    return (emb_long, logits)


# speedup vs baseline: 1.2339x; 1.1496x over previous
import functools

import jax
import jax.numpy as jnp
import numpy as np
from jax import lax
from jax.experimental import pallas as pl
from jax.experimental.pallas import tpu as pltpu
from jax.experimental.pallas import tpu_sc as plsc

N = 10000
E = 160000
D_IN = 128
H1 = 64
H2 = 32
EMB = 16

NC = 2
NS = 16
NW = NC * NS

N_PAD = N + 112
ROWS_PER_TILE = N_PAD // NS
DEG_M = 2 * N + 16
MP_CHUNK = 128
MP_NCHUNK = 40
NCHUNKS = NW * MP_NCHUNK
DEG_VEC = 640


def _stage_a_body(f_ref, w1_ref, o_ref):
    o_ref[...] = jnp.dot(f_ref[...], w1_ref[...],
                         preferred_element_type=jnp.float32)


def _stage_a(features, W1):
    blk = 1000
    return pl.pallas_call(
        _stage_a_body,
        grid=(N // blk,),
        in_specs=[
            pl.BlockSpec((blk, D_IN), lambda i: (i, 0)),
            pl.BlockSpec((D_IN, H1), lambda i: (0, 0)),
        ],
        out_specs=pl.BlockSpec((blk, H1), lambda i: (i, 0)),
        out_shape=jax.ShapeDtypeStruct((N, H1), jnp.float32),
    )(features, W1)


def _scale_body(z_ref, ns_ref, o_ref):
    o_ref[...] = (z_ref[...] * ns_ref[...]).astype(jnp.bfloat16)


def _scale(z1, norm_src):
    blk = 1000
    return pl.pallas_call(
        _scale_body,
        grid=(N // blk,),
        in_specs=[
            pl.BlockSpec((blk, H1), lambda i: (i, 0)),
            pl.BlockSpec((blk, 1), lambda i: (i, 0)),
        ],
        out_specs=pl.BlockSpec((blk, H1), lambda i: (i, 0)),
        out_shape=jax.ShapeDtypeStruct((N, H1), jnp.bfloat16),
    )(z1, norm_src)


def _stage_b_body(m_ref, nd_ref, b1_ref, w2_ref, wext_ref, ns_ref, o_ref):
    m = m_ref[0].astype(jnp.float32) + m_ref[1].astype(jnp.float32)
    x = jnp.maximum(m * nd_ref[...] + b1_ref[...], 0.0)
    w2e = jnp.dot(w2_ref[...], wext_ref[...], preferred_element_type=jnp.float32)
    o_ref[...] = jnp.dot(x, w2e, preferred_element_type=jnp.float32) * ns_ref[...]


def _stage_b(msg1p, norm_dst, b1, W2, Wext, norm_src):
    blk = 1000
    return pl.pallas_call(
        _stage_b_body,
        grid=(N // blk,),
        in_specs=[
            pl.BlockSpec((NC, blk, H1), lambda i: (0, i, 0)),
            pl.BlockSpec((blk, 1), lambda i: (i, 0)),
            pl.BlockSpec((1, H1), lambda i: (0, 0)),
            pl.BlockSpec((H1, H2), lambda i: (0, 0)),
            pl.BlockSpec((H2, EMB), lambda i: (0, 0)),
            pl.BlockSpec((blk, 1), lambda i: (i, 0)),
        ],
        out_specs=pl.BlockSpec((blk, EMB), lambda i: (i, 0)),
        out_shape=jax.ShapeDtypeStruct((N, EMB), jnp.float32),
    )(msg1p, norm_dst, b1, W2, Wext, norm_src)


def _stage_c1_body(m_ref, nd_ref, b2_ref, wext_ref, bext_ref, o_ref):
    b2e = jnp.dot(b2_ref[...], wext_ref[...], preferred_element_type=jnp.float32)
    o_ref[...] = (m_ref[0] + m_ref[1]) * nd_ref[...] + b2e + bext_ref[...]


def _stage_c1(msg2p, norm_dst, b2, Wext, bext):
    blk = 2000
    return pl.pallas_call(
        _stage_c1_body,
        grid=(N // blk,),
        in_specs=[
            pl.BlockSpec((NC, blk, EMB), lambda i: (0, i, 0)),
            pl.BlockSpec((blk, 1), lambda i: (i, 0)),
            pl.BlockSpec((1, H2), lambda i: (0, 0)),
            pl.BlockSpec((H2, EMB), lambda i: (0, 0)),
            pl.BlockSpec((1, EMB), lambda i: (0, 0)),
        ],
        out_specs=pl.BlockSpec((blk, EMB), lambda i: (i, 0)),
        out_shape=jax.ShapeDtypeStruct((N, EMB), jnp.float32),
    )(msg2p, norm_dst, b2, Wext, bext)


def _stage_c2_body(ei_ref, ej_ref, o_ref):
    g = lax.dot_general(ei_ref[...], ej_ref[...],
                        (((1,), (1,)), ((), ())),
                        preferred_element_type=jnp.float32)
    o_ref[...] = 0.5 * jnp.tanh(0.5 * g) + 0.5


def _stage_c2(emb):
    bi, bj = 512, 2048
    gi = (N + bi - 1) // bi
    gj = (N + bj - 1) // bj
    return pl.pallas_call(
        _stage_c2_body,
        grid=(gi, gj),
        in_specs=[
            pl.BlockSpec((bi, EMB), lambda i, j: (i, 0)),
            pl.BlockSpec((bj, EMB), lambda i, j: (j, 0)),
        ],
        out_specs=pl.BlockSpec((bi, bj), lambda i, j: (i, j)),
        out_shape=jax.ShapeDtypeStruct((N, N), jnp.float32),
    )(emb, emb)


def _norms_body(dp_ref, o_ref):
    deg = jnp.sum(dp_ref[...], axis=0, keepdims=True)
    o_ref[...] = lax.rsqrt(jnp.maximum(deg, 1.0))


def _norms(deg_partials):
    p, m = deg_partials.shape
    return pl.pallas_call(
        _norms_body,
        in_specs=[pl.BlockSpec((p, m), lambda: (0, 0))],
        out_specs=pl.BlockSpec((1, m), lambda: (0, 0)),
        out_shape=jax.ShapeDtypeStruct((1, m), jnp.float32),
    )(deg_partials)


_SC_MESH = plsc.VectorSubcoreMesh(core_axis_name="c", subcore_axis_name="s")
_SC_PARAMS = pltpu.CompilerParams(needs_layout_passes=False,
                                  use_tc_tiling_on_sc=False)


@functools.partial(
    pl.kernel,
    out_type=jax.ShapeDtypeStruct((NW, DEG_M), jnp.float32),
    mesh=_SC_MESH,
    compiler_params=_SC_PARAMS,
    scratch_types=[
        pltpu.VMEM((DEG_VEC * 16,), jnp.int32),
        pltpu.VMEM((DEG_M,), jnp.float32),
    ],
)
def _sc_degrees(idx_hbm, out_hbm, idx_v, acc_v):
    c = lax.axis_index("c")
    s = lax.axis_index("s")
    wid = s * NC + c
    pltpu.sync_copy(idx_hbm.at[c].at[s], idx_v)
    zeros16 = jnp.zeros((16,), jnp.float32)

    def zbody(i, carry):
        acc_v[pl.ds(i * 16, 16)] = zeros16
        return carry

    lax.fori_loop(0, DEG_M // 16, zbody, 0)
    ones16 = jnp.ones((16,), jnp.float32)

    def ebody(i, carry):
        v = idx_v[pl.ds(i * 16, 16)]
        plsc.addupdate_scatter(acc_v, [v], ones16)
        return carry

    lax.fori_loop(0, DEG_VEC, ebody, 0)
    pltpu.sync_copy(acc_v, out_hbm.at[wid])


def _make_sc_mp(W, dtype):
    nbuf = 8
    lanes = 16 if dtype == jnp.float32 else 32

    @functools.partial(
        pl.kernel,
        out_type=jax.ShapeDtypeStruct((NC, N_PAD, W), dtype),
        mesh=_SC_MESH,
        compiler_params=_SC_PARAMS,
        scratch_types=[
            pltpu.VMEM((MP_NCHUNK, MP_CHUNK), jnp.int32),
            pltpu.VMEM((MP_NCHUNK, MP_CHUNK), jnp.int32),
            pltpu.VMEM((nbuf, MP_CHUNK, W), dtype),
            pltpu.VMEM_SHARED((N_PAD, W), dtype),
        ] + [pltpu.SemaphoreType.DMA] * (2 * nbuf),
    )
    def mp(z_hbm, src_hbm, dst_hbm, out_hbm,
           src_v, dst_v, gbuf, acc_sh, *sems):
        gsem = sems[:nbuf]
        ssem = sems[nbuf:]
        c = lax.axis_index("c")
        s = lax.axis_index("s")
        pltpu.sync_copy(src_hbm.at[c].at[s], src_v)
        pltpu.sync_copy(dst_hbm.at[c].at[s], dst_v)
        zvec = jnp.zeros((lanes,), dtype)
        wv = W // lanes

        def zbody(i, carry):
            gbuf[0, i // wv, pl.ds((i % wv) * lanes, lanes)] = zvec
            return carry

        lax.fori_loop(0, MP_CHUNK * wv, zbody, 0)
        row0 = s * ROWS_PER_TILE
        chunks = []
        off = 0
        while off < ROWS_PER_TILE:
            sz = min(MP_CHUNK, ROWS_PER_TILE - off)
            chunks.append((off, sz))
            off += sz
        for off, sz in chunks:
            pltpu.sync_copy(gbuf.at[0].at[pl.ds(0, sz)],
                            acc_sh.at[pl.ds(row0 + off, sz)])
        plsc.subcore_barrier()

        def gather(cid, b, src_v):
            return pltpu.async_copy(z_hbm.at[src_v.at[cid]], gbuf.at[b],
                                    gsem[b])

        def scatter(cid, b, dst_v):
            return pltpu.async_copy(gbuf.at[b], acc_sh.at[dst_v.at[cid]],
                                    ssem[b], add=True)

        look = 4

        nchunk = MP_NCHUNK
        gdesc = {cc: gather(cc, cc % nbuf, src_v) for cc in range(look)}
        sdesc = {}
        for cid in range(nchunk):
            gdesc[cid].wait()
            sdesc[cid] = scatter(cid, cid % nbuf, dst_v)
            nxt = cid + look
            if nxt < nchunk:
                if nxt - nbuf in sdesc:
                    sdesc[nxt - nbuf].wait()
                gdesc[nxt] = gather(nxt, nxt % nbuf, src_v)
        for cid in range(nchunk - nbuf, nchunk):
            sdesc[cid].wait()
        plsc.subcore_barrier()
        for off, sz in chunks:
            pltpu.sync_copy(acc_sh.at[pl.ds(row0 + off, sz)],
                            gbuf.at[0].at[pl.ds(0, sz)])
            pltpu.sync_copy(gbuf.at[0].at[pl.ds(0, sz)],
                            out_hbm.at[c].at[pl.ds(row0 + off, sz)])

    return mp


_sc_mp64 = _make_sc_mp(H1, jnp.bfloat16)
_sc_mp16 = _make_sc_mp(EMB, jnp.float32)


def kernel(features, edge_index, W1, b1, W2, b2, Wext, bext):
    src, dst = edge_index[0], edge_index[1]

    pad_e = NCHUNKS * MP_CHUNK - E
    srcf = jnp.concatenate([src, jnp.zeros((pad_e,), jnp.int32)])
    dstf = jnp.concatenate([dst, jnp.full((pad_e,), N, jnp.int32)])
    src_p = srcf.reshape(NC, NS, MP_NCHUNK, MP_CHUNK)
    dst_p = dstf.reshape(NC, NS, MP_NCHUNK, MP_CHUNK)
    nvec_tot = NW * DEG_VEC
    degf = jnp.concatenate(
        [src, dst + N, jnp.full((nvec_tot * 16 - 2 * E,), 2 * N, jnp.int32)])
    deg_idx = degf.reshape(NC, NS, DEG_VEC * 16)

    deg_partials = _sc_degrees(deg_idx)
    norms = _norms(deg_partials)[0]
    norm_src = norms[:N].reshape(N, 1)
    norm_dst = norms[N:2 * N].reshape(N, 1)

    z1 = _stage_a(features, W1)
    z1s = _scale(z1, norm_src)
    p1 = _sc_mp64(z1s, src_p, dst_p)
    z2s = _stage_b(p1, norm_dst, b1.reshape(1, H1), W2, Wext, norm_src)
    p2 = _sc_mp16(z2s, src_p, dst_p)
    emb_long = _stage_c1(p2, norm_dst, b2.reshape(1, H2), Wext,
                         bext.reshape(1, EMB))
    logits = _stage_c2(emb_long)
    return (emb_long, logits)


# speedup vs baseline: 1.3671x; 1.1080x over previous
import functools

import jax
import jax.numpy as jnp
import numpy as np
from jax import lax
from jax.experimental import pallas as pl
from jax.experimental.pallas import tpu as pltpu
from jax.experimental.pallas import tpu_sc as plsc

N = 10000
E = 160000
D_IN = 128
H1 = 64
H2 = 32
EMB = 16

NC = 2
NS = 16
NW = NC * NS

N_PAD = N + 112
ROWS_PER_TILE = N_PAD // NS
DEG_M = 2 * N + 16
MP_CHUNK = 128
MP_NCHUNK = 40
NCHUNKS = NW * MP_NCHUNK
DEG_VEC = 640


def _stage_a_body(f_ref, w1_ref, o_ref):
    o_ref[...] = jnp.dot(f_ref[...], w1_ref[...],
                         preferred_element_type=jnp.float32)


def _stage_a(features, W1):
    blk = 1000
    return pl.pallas_call(
        _stage_a_body,
        grid=(N // blk,),
        in_specs=[
            pl.BlockSpec((blk, D_IN), lambda i: (i, 0)),
            pl.BlockSpec((D_IN, H1), lambda i: (0, 0)),
        ],
        out_specs=pl.BlockSpec((blk, H1), lambda i: (i, 0)),
        out_shape=jax.ShapeDtypeStruct((N, H1), jnp.float32),
    )(features, W1)


def _scale_body(z_ref, ns_ref, o_ref):
    o_ref[...] = (z_ref[...] * ns_ref[...]).astype(jnp.bfloat16)


def _scale(z1, norm_src):
    blk = 1000
    return pl.pallas_call(
        _scale_body,
        grid=(N // blk,),
        in_specs=[
            pl.BlockSpec((blk, H1), lambda i: (i, 0)),
            pl.BlockSpec((blk, 1), lambda i: (i, 0)),
        ],
        out_specs=pl.BlockSpec((blk, H1), lambda i: (i, 0)),
        out_shape=jax.ShapeDtypeStruct((N, H1), jnp.bfloat16),
    )(z1, norm_src)


def _stage_b_body(m_ref, nd_ref, b1_ref, w2_ref, wext_ref, ns_ref, o_ref):
    m = m_ref[0].astype(jnp.float32) + m_ref[1].astype(jnp.float32)
    x = jnp.maximum(m * nd_ref[...] + b1_ref[...], 0.0)
    w2e = jnp.dot(w2_ref[...], wext_ref[...], preferred_element_type=jnp.float32)
    o_ref[...] = jnp.dot(x, w2e, preferred_element_type=jnp.float32) * ns_ref[...]


def _stage_b(msg1p, norm_dst, b1, W2, Wext, norm_src):
    blk = 1000
    return pl.pallas_call(
        _stage_b_body,
        grid=(N // blk,),
        in_specs=[
            pl.BlockSpec((NC, blk, H1), lambda i: (0, i, 0)),
            pl.BlockSpec((blk, 1), lambda i: (i, 0)),
            pl.BlockSpec((1, H1), lambda i: (0, 0)),
            pl.BlockSpec((H1, H2), lambda i: (0, 0)),
            pl.BlockSpec((H2, EMB), lambda i: (0, 0)),
            pl.BlockSpec((blk, 1), lambda i: (i, 0)),
        ],
        out_specs=pl.BlockSpec((blk, EMB), lambda i: (i, 0)),
        out_shape=jax.ShapeDtypeStruct((N, EMB), jnp.float32),
    )(msg1p, norm_dst, b1, W2, Wext, norm_src)


def _stage_c1_body(m_ref, nd_ref, b2_ref, wext_ref, bext_ref, o_ref):
    b2e = jnp.dot(b2_ref[...], wext_ref[...], preferred_element_type=jnp.float32)
    o_ref[...] = (m_ref[0] + m_ref[1]) * nd_ref[...] + b2e + bext_ref[...]


def _stage_c1(msg2p, norm_dst, b2, Wext, bext):
    blk = 2000
    return pl.pallas_call(
        _stage_c1_body,
        grid=(N // blk,),
        in_specs=[
            pl.BlockSpec((NC, blk, EMB), lambda i: (0, i, 0)),
            pl.BlockSpec((blk, 1), lambda i: (i, 0)),
            pl.BlockSpec((1, H2), lambda i: (0, 0)),
            pl.BlockSpec((H2, EMB), lambda i: (0, 0)),
            pl.BlockSpec((1, EMB), lambda i: (0, 0)),
        ],
        out_specs=pl.BlockSpec((blk, EMB), lambda i: (i, 0)),
        out_shape=jax.ShapeDtypeStruct((N, EMB), jnp.float32),
    )(msg2p, norm_dst, b2, Wext, bext)


def _stage_c2_body(ei_ref, ej_ref, o_ref):
    g = lax.dot_general(ei_ref[...], ej_ref[...],
                        (((1,), (1,)), ((), ())),
                        preferred_element_type=jnp.float32)
    o_ref[...] = 0.5 * jnp.tanh(0.5 * g) + 0.5


def _stage_c2(emb):
    bi, bj = 1024, 2048
    gi = (N + bi - 1) // bi
    gj = (N + bj - 1) // bj
    return pl.pallas_call(
        _stage_c2_body,
        grid=(gi, gj),
        in_specs=[
            pl.BlockSpec((bi, EMB), lambda i, j: (i, 0)),
            pl.BlockSpec((bj, EMB), lambda i, j: (j, 0)),
        ],
        out_specs=pl.BlockSpec((bi, bj), lambda i, j: (i, j)),
        out_shape=jax.ShapeDtypeStruct((N, N), jnp.float32),
    )(emb, emb)


def _norms_body(dp_ref, o_ref):
    deg = jnp.sum(dp_ref[...], axis=0, keepdims=True)
    o_ref[...] = lax.rsqrt(jnp.maximum(deg, 1.0))


def _norms(deg_partials):
    p, m = deg_partials.shape
    return pl.pallas_call(
        _norms_body,
        in_specs=[pl.BlockSpec((p, m), lambda: (0, 0))],
        out_specs=pl.BlockSpec((1, m), lambda: (0, 0)),
        out_shape=jax.ShapeDtypeStruct((1, m), jnp.float32),
    )(deg_partials)


_SC_MESH = plsc.VectorSubcoreMesh(core_axis_name="c", subcore_axis_name="s")
_SC_PARAMS = pltpu.CompilerParams(needs_layout_passes=False,
                                  use_tc_tiling_on_sc=False)


@functools.partial(
    pl.kernel,
    out_type=jax.ShapeDtypeStruct((NW, DEG_M), jnp.float32),
    mesh=_SC_MESH,
    compiler_params=_SC_PARAMS,
    scratch_types=[
        pltpu.VMEM((DEG_VEC * 16,), jnp.int32),
        pltpu.VMEM((DEG_M,), jnp.float32),
    ],
)
def _sc_degrees(idx_hbm, out_hbm, idx_v, acc_v):
    c = lax.axis_index("c")
    s = lax.axis_index("s")
    wid = s * NC + c
    pltpu.sync_copy(idx_hbm.at[c].at[s], idx_v)
    zeros16 = jnp.zeros((16,), jnp.float32)

    def zbody(i, carry):
        acc_v[pl.ds(i * 16, 16)] = zeros16
        return carry

    lax.fori_loop(0, DEG_M // 16, zbody, 0)
    ones16 = jnp.ones((16,), jnp.float32)

    def ebody(i, carry):
        v = idx_v[pl.ds(i * 16, 16)]
        plsc.addupdate_scatter(acc_v, [v], ones16)
        return carry

    lax.fori_loop(0, DEG_VEC, ebody, 0)
    pltpu.sync_copy(acc_v, out_hbm.at[wid])


def _make_sc_mp(W, dtype):
    nbuf = 8
    lanes = 16 if dtype == jnp.float32 else 32

    @functools.partial(
        pl.kernel,
        out_type=jax.ShapeDtypeStruct((NC, N_PAD, W), dtype),
        mesh=_SC_MESH,
        compiler_params=_SC_PARAMS,
        scratch_types=[
            pltpu.VMEM((MP_NCHUNK, MP_CHUNK), jnp.int32),
            pltpu.VMEM((MP_NCHUNK, MP_CHUNK), jnp.int32),
            pltpu.VMEM((nbuf, MP_CHUNK, W), dtype),
            pltpu.VMEM_SHARED((N_PAD, W), dtype),
        ] + [pltpu.SemaphoreType.DMA] * (2 * nbuf),
    )
    def mp(z_hbm, src_hbm, dst_hbm, out_hbm,
           src_v, dst_v, gbuf, acc_sh, *sems):
        gsem = sems[:nbuf]
        ssem = sems[nbuf:]
        c = lax.axis_index("c")
        s = lax.axis_index("s")
        pltpu.sync_copy(src_hbm.at[c].at[s], src_v)
        pltpu.sync_copy(dst_hbm.at[c].at[s], dst_v)
        zvec = jnp.zeros((lanes,), dtype)
        wv = W // lanes

        def zbody(i, carry):
            gbuf[0, i // wv, pl.ds((i % wv) * lanes, lanes)] = zvec
            return carry

        lax.fori_loop(0, MP_CHUNK * wv, zbody, 0)
        row0 = s * ROWS_PER_TILE
        chunks = []
        off = 0
        while off < ROWS_PER_TILE:
            sz = min(MP_CHUNK, ROWS_PER_TILE - off)
            chunks.append((off, sz))
            off += sz
        for off, sz in chunks:
            pltpu.sync_copy(gbuf.at[0].at[pl.ds(0, sz)],
                            acc_sh.at[pl.ds(row0 + off, sz)])
        plsc.subcore_barrier()

        def gather(cid, b, src_v):
            return pltpu.async_copy(z_hbm.at[src_v.at[cid]], gbuf.at[b],
                                    gsem[b])

        def scatter(cid, b, dst_v):
            return pltpu.async_copy(gbuf.at[b], acc_sh.at[dst_v.at[cid]],
                                    ssem[b], add=True)

        look = 4

        nchunk = MP_NCHUNK
        gdesc = {cc: gather(cc, cc % nbuf, src_v) for cc in range(look)}
        sdesc = {}
        for cid in range(nchunk):
            gdesc[cid].wait()
            sdesc[cid] = scatter(cid, cid % nbuf, dst_v)
            nxt = cid + look
            if nxt < nchunk:
                if nxt - nbuf in sdesc:
                    sdesc[nxt - nbuf].wait()
                gdesc[nxt] = gather(nxt, nxt % nbuf, src_v)
        for cid in range(nchunk - nbuf, nchunk):
            sdesc[cid].wait()
        plsc.subcore_barrier()
        for off, sz in chunks:
            pltpu.sync_copy(acc_sh.at[pl.ds(row0 + off, sz)],
                            gbuf.at[0].at[pl.ds(0, sz)])
            pltpu.sync_copy(gbuf.at[0].at[pl.ds(0, sz)],
                            out_hbm.at[c].at[pl.ds(row0 + off, sz)])

    return mp


_sc_mp64 = _make_sc_mp(H1, jnp.bfloat16)
_sc_mp16 = _make_sc_mp(EMB, jnp.float32)


def kernel(features, edge_index, W1, b1, W2, b2, Wext, bext):
    src, dst = edge_index[0], edge_index[1]

    pad_e = NCHUNKS * MP_CHUNK - E
    srcf = jnp.concatenate([src, jnp.zeros((pad_e,), jnp.int32)])
    dstf = jnp.concatenate([dst, jnp.full((pad_e,), N, jnp.int32)])
    src_p = srcf.reshape(NC, NS, MP_NCHUNK, MP_CHUNK)
    dst_p = dstf.reshape(NC, NS, MP_NCHUNK, MP_CHUNK)
    nvec_tot = NW * DEG_VEC
    degf = jnp.concatenate(
        [src, dst + N, jnp.full((nvec_tot * 16 - 2 * E,), 2 * N, jnp.int32)])
    deg_idx = degf.reshape(NC, NS, DEG_VEC * 16)

    deg_partials = _sc_degrees(deg_idx)
    norms = _norms(deg_partials)[0]
    norm_src = norms[:N].reshape(N, 1)
    norm_dst = norms[N:2 * N].reshape(N, 1)

    z1 = _stage_a(features, W1)
    z1s = _scale(z1, norm_src)
    p1 = _sc_mp64(z1s, src_p, dst_p)
    z2s = _stage_b(p1, norm_dst, b1.reshape(1, H1), W2, Wext, norm_src)
    p2 = _sc_mp16(z2s, src_p, dst_p)
    emb_long = _stage_c1(p2, norm_dst, b2.reshape(1, H2), Wext,
                         bext.reshape(1, EMB))
    logits = _stage_c2(emb_long)
    return (emb_long, logits)


# speedup vs baseline: 1.4169x; 1.0364x over previous
import functools

import jax
import jax.numpy as jnp
import numpy as np
from jax import lax
from jax.experimental import pallas as pl
from jax.experimental.pallas import tpu as pltpu
from jax.experimental.pallas import tpu_sc as plsc

N = 10000
E = 160000
D_IN = 128
H1 = 64
H2 = 32
EMB = 16

NC = 2
NS = 16
NW = NC * NS

N_PAD = N + 112
ROWS_PER_TILE = N_PAD // NS
DEG_M = 2 * N + 16
MP_CHUNK = 128
MP_NCHUNK = 40
NCHUNKS = NW * MP_NCHUNK
DEG_VEC = 640


def _stage_a_body(f_ref, w1_ref, o_ref):
    o_ref[...] = jnp.dot(f_ref[...], w1_ref[...],
                         preferred_element_type=jnp.float32)


def _stage_a(features, W1):
    blk = 1000
    return pl.pallas_call(
        _stage_a_body,
        grid=(N // blk,),
        in_specs=[
            pl.BlockSpec((blk, D_IN), lambda i: (i, 0)),
            pl.BlockSpec((D_IN, H1), lambda i: (0, 0)),
        ],
        out_specs=pl.BlockSpec((blk, H1), lambda i: (i, 0)),
        out_shape=jax.ShapeDtypeStruct((N, H1), jnp.float32),
    )(features, W1)


def _scale_body(z_ref, ns_ref, o_ref):
    o_ref[...] = (z_ref[...] * ns_ref[...]).astype(jnp.bfloat16)


def _scale(z1, norm_src):
    blk = 1000
    return pl.pallas_call(
        _scale_body,
        grid=(N // blk,),
        in_specs=[
            pl.BlockSpec((blk, H1), lambda i: (i, 0)),
            pl.BlockSpec((blk, 1), lambda i: (i, 0)),
        ],
        out_specs=pl.BlockSpec((blk, H1), lambda i: (i, 0)),
        out_shape=jax.ShapeDtypeStruct((N, H1), jnp.bfloat16),
    )(z1, norm_src)


def _stage_b_body(m_ref, nd_ref, b1_ref, w2_ref, wext_ref, ns_ref, o_ref):
    m = m_ref[0].astype(jnp.float32) + m_ref[1].astype(jnp.float32)
    x = jnp.maximum(m * nd_ref[...] + b1_ref[...], 0.0)
    w2e = jnp.dot(w2_ref[...], wext_ref[...], preferred_element_type=jnp.float32)
    o_ref[...] = jnp.dot(x, w2e, preferred_element_type=jnp.float32) * ns_ref[...]


def _stage_b(msg1p, norm_dst, b1, W2, Wext, norm_src):
    blk = 1000
    return pl.pallas_call(
        _stage_b_body,
        grid=(N // blk,),
        in_specs=[
            pl.BlockSpec((NC, blk, H1), lambda i: (0, i, 0)),
            pl.BlockSpec((blk, 1), lambda i: (i, 0)),
            pl.BlockSpec((1, H1), lambda i: (0, 0)),
            pl.BlockSpec((H1, H2), lambda i: (0, 0)),
            pl.BlockSpec((H2, EMB), lambda i: (0, 0)),
            pl.BlockSpec((blk, 1), lambda i: (i, 0)),
        ],
        out_specs=pl.BlockSpec((blk, EMB), lambda i: (i, 0)),
        out_shape=jax.ShapeDtypeStruct((N, EMB), jnp.float32),
    )(msg1p, norm_dst, b1, W2, Wext, norm_src)


def _stage_c1_body(m_ref, nd_ref, b2_ref, wext_ref, bext_ref, o_ref):
    b2e = jnp.dot(b2_ref[...], wext_ref[...], preferred_element_type=jnp.float32)
    o_ref[...] = (m_ref[0] + m_ref[1]) * nd_ref[...] + b2e + bext_ref[...]


def _stage_c1(msg2p, norm_dst, b2, Wext, bext):
    blk = 2000
    return pl.pallas_call(
        _stage_c1_body,
        grid=(N // blk,),
        in_specs=[
            pl.BlockSpec((NC, blk, EMB), lambda i: (0, i, 0)),
            pl.BlockSpec((blk, 1), lambda i: (i, 0)),
            pl.BlockSpec((1, H2), lambda i: (0, 0)),
            pl.BlockSpec((H2, EMB), lambda i: (0, 0)),
            pl.BlockSpec((1, EMB), lambda i: (0, 0)),
        ],
        out_specs=pl.BlockSpec((blk, EMB), lambda i: (i, 0)),
        out_shape=jax.ShapeDtypeStruct((N, EMB), jnp.float32),
    )(msg2p, norm_dst, b2, Wext, bext)


def _stage_c2_body(ei_ref, ej_ref, o_ref):
    g = lax.dot_general(ei_ref[...], ej_ref[...],
                        (((1,), (1,)), ((), ())),
                        preferred_element_type=jnp.float32)
    o_ref[...] = 0.5 * jnp.tanh(0.5 * g) + 0.5


def _stage_c2(emb):
    bi, bj = 2048, 2048
    gi = (N + bi - 1) // bi
    gj = (N + bj - 1) // bj
    return pl.pallas_call(
        _stage_c2_body,
        grid=(gi, gj),
        in_specs=[
            pl.BlockSpec((bi, EMB), lambda i, j: (i, 0)),
            pl.BlockSpec((bj, EMB), lambda i, j: (j, 0)),
        ],
        out_specs=pl.BlockSpec((bi, bj), lambda i, j: (i, j)),
        out_shape=jax.ShapeDtypeStruct((N, N), jnp.float32),
    )(emb, emb)


def _norms_body(dp_ref, o_ref):
    deg = jnp.sum(dp_ref[...], axis=0, keepdims=True)
    o_ref[...] = lax.rsqrt(jnp.maximum(deg, 1.0))


def _norms(deg_partials):
    p, m = deg_partials.shape
    return pl.pallas_call(
        _norms_body,
        in_specs=[pl.BlockSpec((p, m), lambda: (0, 0))],
        out_specs=pl.BlockSpec((1, m), lambda: (0, 0)),
        out_shape=jax.ShapeDtypeStruct((1, m), jnp.float32),
    )(deg_partials)


_SC_MESH = plsc.VectorSubcoreMesh(core_axis_name="c", subcore_axis_name="s")
_SC_PARAMS = pltpu.CompilerParams(needs_layout_passes=False,
                                  use_tc_tiling_on_sc=False)


@functools.partial(
    pl.kernel,
    out_type=jax.ShapeDtypeStruct((NW, DEG_M), jnp.float32),
    mesh=_SC_MESH,
    compiler_params=_SC_PARAMS,
    scratch_types=[
        pltpu.VMEM((DEG_VEC * 16,), jnp.int32),
        pltpu.VMEM((DEG_M,), jnp.float32),
    ],
)
def _sc_degrees(idx_hbm, out_hbm, idx_v, acc_v):
    c = lax.axis_index("c")
    s = lax.axis_index("s")
    wid = s * NC + c
    pltpu.sync_copy(idx_hbm.at[c].at[s], idx_v)
    zeros16 = jnp.zeros((16,), jnp.float32)

    def zbody(i, carry):
        acc_v[pl.ds(i * 16, 16)] = zeros16
        return carry

    lax.fori_loop(0, DEG_M // 16, zbody, 0)
    ones16 = jnp.ones((16,), jnp.float32)

    def ebody(i, carry):
        v = idx_v[pl.ds(i * 16, 16)]
        plsc.addupdate_scatter(acc_v, [v], ones16)
        return carry

    lax.fori_loop(0, DEG_VEC, ebody, 0)
    pltpu.sync_copy(acc_v, out_hbm.at[wid])


def _make_sc_mp(W, dtype):
    nbuf = 8
    lanes = 16 if dtype == jnp.float32 else 32

    @functools.partial(
        pl.kernel,
        out_type=jax.ShapeDtypeStruct((NC, N_PAD, W), dtype),
        mesh=_SC_MESH,
        compiler_params=_SC_PARAMS,
        scratch_types=[
            pltpu.VMEM((MP_NCHUNK, MP_CHUNK), jnp.int32),
            pltpu.VMEM((MP_NCHUNK, MP_CHUNK), jnp.int32),
            pltpu.VMEM((nbuf, MP_CHUNK, W), dtype),
            pltpu.VMEM_SHARED((N_PAD, W), dtype),
        ] + [pltpu.SemaphoreType.DMA] * (2 * nbuf),
    )
    def mp(z_hbm, src_hbm, dst_hbm, out_hbm,
           src_v, dst_v, gbuf, acc_sh, *sems):
        gsem = sems[:nbuf]
        ssem = sems[nbuf:]
        c = lax.axis_index("c")
        s = lax.axis_index("s")
        pltpu.sync_copy(src_hbm.at[c].at[s], src_v)
        pltpu.sync_copy(dst_hbm.at[c].at[s], dst_v)
        zvec = jnp.zeros((lanes,), dtype)
        wv = W // lanes

        def zbody(i, carry):
            gbuf[0, i // wv, pl.ds((i % wv) * lanes, lanes)] = zvec
            return carry

        lax.fori_loop(0, MP_CHUNK * wv, zbody, 0)
        row0 = s * ROWS_PER_TILE
        chunks = []
        off = 0
        while off < ROWS_PER_TILE:
            sz = min(MP_CHUNK, ROWS_PER_TILE - off)
            chunks.append((off, sz))
            off += sz
        for off, sz in chunks:
            pltpu.sync_copy(gbuf.at[0].at[pl.ds(0, sz)],
                            acc_sh.at[pl.ds(row0 + off, sz)])
        plsc.subcore_barrier()

        def gather(cid, b, src_v):
            return pltpu.async_copy(z_hbm.at[src_v.at[cid]], gbuf.at[b],
                                    gsem[b])

        def scatter(cid, b, dst_v):
            return pltpu.async_copy(gbuf.at[b], acc_sh.at[dst_v.at[cid]],
                                    ssem[b], add=True)

        look = 4

        nchunk = MP_NCHUNK
        gdesc = {cc: gather(cc, cc % nbuf, src_v) for cc in range(look)}
        sdesc = {}
        for cid in range(nchunk):
            gdesc[cid].wait()
            sdesc[cid] = scatter(cid, cid % nbuf, dst_v)
            nxt = cid + look
            if nxt < nchunk:
                if nxt - nbuf in sdesc:
                    sdesc[nxt - nbuf].wait()
                gdesc[nxt] = gather(nxt, nxt % nbuf, src_v)
        for cid in range(nchunk - nbuf, nchunk):
            sdesc[cid].wait()
        plsc.subcore_barrier()
        for off, sz in chunks:
            pltpu.sync_copy(acc_sh.at[pl.ds(row0 + off, sz)],
                            gbuf.at[0].at[pl.ds(0, sz)])
            pltpu.sync_copy(gbuf.at[0].at[pl.ds(0, sz)],
                            out_hbm.at[c].at[pl.ds(row0 + off, sz)])

    return mp


_sc_mp64 = _make_sc_mp(H1, jnp.bfloat16)
_sc_mp16 = _make_sc_mp(EMB, jnp.float32)


def kernel(features, edge_index, W1, b1, W2, b2, Wext, bext):
    src, dst = edge_index[0], edge_index[1]

    pad_e = NCHUNKS * MP_CHUNK - E
    srcf = jnp.concatenate([src, jnp.zeros((pad_e,), jnp.int32)])
    dstf = jnp.concatenate([dst, jnp.full((pad_e,), N, jnp.int32)])
    src_p = srcf.reshape(NC, NS, MP_NCHUNK, MP_CHUNK)
    dst_p = dstf.reshape(NC, NS, MP_NCHUNK, MP_CHUNK)
    nvec_tot = NW * DEG_VEC
    degf = jnp.concatenate(
        [src, dst + N, jnp.full((nvec_tot * 16 - 2 * E,), 2 * N, jnp.int32)])
    deg_idx = degf.reshape(NC, NS, DEG_VEC * 16)

    deg_partials = _sc_degrees(deg_idx)
    norms = _norms(deg_partials)[0]
    norm_src = norms[:N].reshape(N, 1)
    norm_dst = norms[N:2 * N].reshape(N, 1)

    z1 = _stage_a(features, W1)
    z1s = _scale(z1, norm_src)
    p1 = _sc_mp64(z1s, src_p, dst_p)
    z2s = _stage_b(p1, norm_dst, b1.reshape(1, H1), W2, Wext, norm_src)
    p2 = _sc_mp16(z2s, src_p, dst_p)
    emb_long = _stage_c1(p2, norm_dst, b2.reshape(1, H2), Wext,
                         bext.reshape(1, EMB))
    logits = _stage_c2(emb_long)
    return (emb_long, logits)


# speedup vs baseline: 1.4254x; 1.0060x over previous
import functools

import jax
import jax.numpy as jnp
import numpy as np
from jax import lax
from jax.experimental import pallas as pl
from jax.experimental.pallas import tpu as pltpu
from jax.experimental.pallas import tpu_sc as plsc

N = 10000
E = 160000
D_IN = 128
H1 = 64
H2 = 32
EMB = 16

NC = 2
NS = 16
NW = NC * NS

N_PAD = N + 112
ROWS_PER_TILE = N_PAD // NS
DEG_M = 2 * N + 16
MP_CHUNK = 128
MP_NCHUNK = 40
NCHUNKS = NW * MP_NCHUNK
DEG_VEC = 640


def _stage_a_body(f_ref, w1_ref, o_ref):
    o_ref[...] = jnp.dot(f_ref[...], w1_ref[...],
                         preferred_element_type=jnp.float32)


def _stage_a(features, W1):
    blk = 1000
    return pl.pallas_call(
        _stage_a_body,
        grid=(N // blk,),
        in_specs=[
            pl.BlockSpec((blk, D_IN), lambda i: (i, 0)),
            pl.BlockSpec((D_IN, H1), lambda i: (0, 0)),
        ],
        out_specs=pl.BlockSpec((blk, H1), lambda i: (i, 0)),
        out_shape=jax.ShapeDtypeStruct((N, H1), jnp.float32),
    )(features, W1)


def _scale_body(z_ref, ns_ref, o_ref):
    o_ref[...] = (z_ref[...] * ns_ref[...]).astype(jnp.bfloat16)


def _scale(z1, norm_src):
    blk = 1000
    return pl.pallas_call(
        _scale_body,
        grid=(N // blk,),
        in_specs=[
            pl.BlockSpec((blk, H1), lambda i: (i, 0)),
            pl.BlockSpec((blk, 1), lambda i: (i, 0)),
        ],
        out_specs=pl.BlockSpec((blk, H1), lambda i: (i, 0)),
        out_shape=jax.ShapeDtypeStruct((N, H1), jnp.bfloat16),
    )(z1, norm_src)


def _stage_b_body(m_ref, nd_ref, b1_ref, w2_ref, wext_ref, ns_ref, o_ref):
    m = m_ref[0].astype(jnp.float32) + m_ref[1].astype(jnp.float32)
    x = jnp.maximum(m * nd_ref[...] + b1_ref[...], 0.0)
    w2e = jnp.dot(w2_ref[...], wext_ref[...], preferred_element_type=jnp.float32)
    o_ref[...] = jnp.dot(x, w2e, preferred_element_type=jnp.float32) * ns_ref[...]


def _stage_b(msg1p, norm_dst, b1, W2, Wext, norm_src):
    blk = 1000
    return pl.pallas_call(
        _stage_b_body,
        grid=(N // blk,),
        in_specs=[
            pl.BlockSpec((NC, blk, H1), lambda i: (0, i, 0)),
            pl.BlockSpec((blk, 1), lambda i: (i, 0)),
            pl.BlockSpec((1, H1), lambda i: (0, 0)),
            pl.BlockSpec((H1, H2), lambda i: (0, 0)),
            pl.BlockSpec((H2, EMB), lambda i: (0, 0)),
            pl.BlockSpec((blk, 1), lambda i: (i, 0)),
        ],
        out_specs=pl.BlockSpec((blk, EMB), lambda i: (i, 0)),
        out_shape=jax.ShapeDtypeStruct((N, EMB), jnp.float32),
    )(msg1p, norm_dst, b1, W2, Wext, norm_src)


def _stage_c1_body(m_ref, nd_ref, b2_ref, wext_ref, bext_ref, o_ref):
    b2e = jnp.dot(b2_ref[...], wext_ref[...], preferred_element_type=jnp.float32)
    o_ref[...] = (m_ref[0] + m_ref[1]) * nd_ref[...] + b2e + bext_ref[...]


def _stage_c1(msg2p, norm_dst, b2, Wext, bext):
    blk = 2000
    return pl.pallas_call(
        _stage_c1_body,
        grid=(N // blk,),
        in_specs=[
            pl.BlockSpec((NC, blk, EMB), lambda i: (0, i, 0)),
            pl.BlockSpec((blk, 1), lambda i: (i, 0)),
            pl.BlockSpec((1, H2), lambda i: (0, 0)),
            pl.BlockSpec((H2, EMB), lambda i: (0, 0)),
            pl.BlockSpec((1, EMB), lambda i: (0, 0)),
        ],
        out_specs=pl.BlockSpec((blk, EMB), lambda i: (i, 0)),
        out_shape=jax.ShapeDtypeStruct((N, EMB), jnp.float32),
    )(msg2p, norm_dst, b2, Wext, bext)


def _stage_c2_body(ei_ref, ej_ref, o_ref):
    g = lax.dot_general(ei_ref[...], ej_ref[...],
                        (((1,), (1,)), ((), ())),
                        preferred_element_type=jnp.float32)
    o_ref[...] = 0.5 * jnp.tanh(0.5 * g) + 0.5


def _stage_c2(emb):
    bi, bj = 2560, 2048
    gi = (N + bi - 1) // bi
    gj = (N + bj - 1) // bj
    return pl.pallas_call(
        _stage_c2_body,
        grid=(gi, gj),
        in_specs=[
            pl.BlockSpec((bi, EMB), lambda i, j: (i, 0)),
            pl.BlockSpec((bj, EMB), lambda i, j: (j, 0)),
        ],
        out_specs=pl.BlockSpec((bi, bj), lambda i, j: (i, j)),
        out_shape=jax.ShapeDtypeStruct((N, N), jnp.float32),
    )(emb, emb)


def _norms_body(dp_ref, o_ref):
    deg = jnp.sum(dp_ref[...], axis=0, keepdims=True)
    o_ref[...] = lax.rsqrt(jnp.maximum(deg, 1.0))


def _norms(deg_partials):
    p, m = deg_partials.shape
    return pl.pallas_call(
        _norms_body,
        in_specs=[pl.BlockSpec((p, m), lambda: (0, 0))],
        out_specs=pl.BlockSpec((1, m), lambda: (0, 0)),
        out_shape=jax.ShapeDtypeStruct((1, m), jnp.float32),
    )(deg_partials)


_SC_MESH = plsc.VectorSubcoreMesh(core_axis_name="c", subcore_axis_name="s")
_SC_PARAMS = pltpu.CompilerParams(needs_layout_passes=False,
                                  use_tc_tiling_on_sc=False)


@functools.partial(
    pl.kernel,
    out_type=jax.ShapeDtypeStruct((NW, DEG_M), jnp.float32),
    mesh=_SC_MESH,
    compiler_params=_SC_PARAMS,
    scratch_types=[
        pltpu.VMEM((DEG_VEC * 16,), jnp.int32),
        pltpu.VMEM((DEG_M,), jnp.float32),
    ],
)
def _sc_degrees(idx_hbm, out_hbm, idx_v, acc_v):
    c = lax.axis_index("c")
    s = lax.axis_index("s")
    wid = s * NC + c
    pltpu.sync_copy(idx_hbm.at[c].at[s], idx_v)
    zeros16 = jnp.zeros((16,), jnp.float32)

    def zbody(i, carry):
        acc_v[pl.ds(i * 16, 16)] = zeros16
        return carry

    lax.fori_loop(0, DEG_M // 16, zbody, 0)
    ones16 = jnp.ones((16,), jnp.float32)

    def ebody(i, carry):
        v = idx_v[pl.ds(i * 16, 16)]
        plsc.addupdate_scatter(acc_v, [v], ones16)
        return carry

    lax.fori_loop(0, DEG_VEC, ebody, 0)
    pltpu.sync_copy(acc_v, out_hbm.at[wid])


def _make_sc_mp(W, dtype):
    nbuf = 8
    lanes = 16 if dtype == jnp.float32 else 32

    @functools.partial(
        pl.kernel,
        out_type=jax.ShapeDtypeStruct((NC, N_PAD, W), dtype),
        mesh=_SC_MESH,
        compiler_params=_SC_PARAMS,
        scratch_types=[
            pltpu.VMEM((MP_NCHUNK, MP_CHUNK), jnp.int32),
            pltpu.VMEM((MP_NCHUNK, MP_CHUNK), jnp.int32),
            pltpu.VMEM((nbuf, MP_CHUNK, W), dtype),
            pltpu.VMEM_SHARED((N_PAD, W), dtype),
        ] + [pltpu.SemaphoreType.DMA] * (2 * nbuf),
    )
    def mp(z_hbm, src_hbm, dst_hbm, out_hbm,
           src_v, dst_v, gbuf, acc_sh, *sems):
        gsem = sems[:nbuf]
        ssem = sems[nbuf:]
        c = lax.axis_index("c")
        s = lax.axis_index("s")
        pltpu.sync_copy(src_hbm.at[c].at[s], src_v)
        pltpu.sync_copy(dst_hbm.at[c].at[s], dst_v)
        zvec = jnp.zeros((lanes,), dtype)
        wv = W // lanes

        def zbody(i, carry):
            gbuf[0, i // wv, pl.ds((i % wv) * lanes, lanes)] = zvec
            return carry

        lax.fori_loop(0, MP_CHUNK * wv, zbody, 0)
        row0 = s * ROWS_PER_TILE
        chunks = []
        off = 0
        while off < ROWS_PER_TILE:
            sz = min(MP_CHUNK, ROWS_PER_TILE - off)
            chunks.append((off, sz))
            off += sz
        for off, sz in chunks:
            pltpu.sync_copy(gbuf.at[0].at[pl.ds(0, sz)],
                            acc_sh.at[pl.ds(row0 + off, sz)])
        plsc.subcore_barrier()

        def gather(cid, b, src_v):
            return pltpu.async_copy(z_hbm.at[src_v.at[cid]], gbuf.at[b],
                                    gsem[b])

        def scatter(cid, b, dst_v):
            return pltpu.async_copy(gbuf.at[b], acc_sh.at[dst_v.at[cid]],
                                    ssem[b], add=True)

        look = 4

        nchunk = MP_NCHUNK
        gdesc = {cc: gather(cc, cc % nbuf, src_v) for cc in range(look)}
        sdesc = {}
        for cid in range(nchunk):
            gdesc[cid].wait()
            sdesc[cid] = scatter(cid, cid % nbuf, dst_v)
            nxt = cid + look
            if nxt < nchunk:
                if nxt - nbuf in sdesc:
                    sdesc[nxt - nbuf].wait()
                gdesc[nxt] = gather(nxt, nxt % nbuf, src_v)
        for cid in range(nchunk - nbuf, nchunk):
            sdesc[cid].wait()
        plsc.subcore_barrier()
        for off, sz in chunks:
            pltpu.sync_copy(acc_sh.at[pl.ds(row0 + off, sz)],
                            gbuf.at[0].at[pl.ds(0, sz)])
            pltpu.sync_copy(gbuf.at[0].at[pl.ds(0, sz)],
                            out_hbm.at[c].at[pl.ds(row0 + off, sz)])

    return mp


_sc_mp64 = _make_sc_mp(H1, jnp.bfloat16)
_sc_mp16 = _make_sc_mp(EMB, jnp.float32)


def kernel(features, edge_index, W1, b1, W2, b2, Wext, bext):
    src, dst = edge_index[0], edge_index[1]

    pad_e = NCHUNKS * MP_CHUNK - E
    srcf = jnp.concatenate([src, jnp.zeros((pad_e,), jnp.int32)])
    dstf = jnp.concatenate([dst, jnp.full((pad_e,), N, jnp.int32)])
    src_p = srcf.reshape(NC, NS, MP_NCHUNK, MP_CHUNK)
    dst_p = dstf.reshape(NC, NS, MP_NCHUNK, MP_CHUNK)
    nvec_tot = NW * DEG_VEC
    degf = jnp.concatenate(
        [src, dst + N, jnp.full((nvec_tot * 16 - 2 * E,), 2 * N, jnp.int32)])
    deg_idx = degf.reshape(NC, NS, DEG_VEC * 16)

    deg_partials = _sc_degrees(deg_idx)
    norms = _norms(deg_partials)[0]
    norm_src = norms[:N].reshape(N, 1)
    norm_dst = norms[N:2 * N].reshape(N, 1)

    z1 = _stage_a(features, W1)
    z1s = _scale(z1, norm_src)
    p1 = _sc_mp64(z1s, src_p, dst_p)
    z2s = _stage_b(p1, norm_dst, b1.reshape(1, H1), W2, Wext, norm_src)
    p2 = _sc_mp16(z2s, src_p, dst_p)
    emb_long = _stage_c1(p2, norm_dst, b2.reshape(1, H2), Wext,
                         bext.reshape(1, EMB))
    logits = _stage_c2(emb_long)
    return (emb_long, logits)
